# baseline (speedup 1.0000x reference)
import jax
import jax.numpy as jnp
from jax.experimental import pallas as pl

_N_ROWS = 16384
_N_CLASSES = 1000
_EPS = 0.001
_ON = 1.0 / _N_CLASSES + _EPS
_OFF = 1.0 / _N_CLASSES - _EPS / (_N_CLASSES - 1)

_BLOCK_ROWS = 256


def _body(x_ref, o_ref):
    x = x_ref[...]
    rowmax = jnp.max(x, axis=1, keepdims=True)
    cols = jax.lax.broadcasted_iota(jnp.int32, x.shape, 1)
    big = jnp.int32(_N_CLASSES)
    amax = jnp.min(jnp.where(x == rowmax, cols, big), axis=1, keepdims=True)
    o_ref[...] = jnp.where(cols == amax, jnp.float32(_ON), jnp.float32(_OFF))


def kernel(input):
    grid = _N_ROWS // _BLOCK_ROWS
    return pl.pallas_call(
        _body,
        grid=(grid,),
        in_specs=[pl.BlockSpec((_BLOCK_ROWS, _N_CLASSES), lambda i: (i, 0))],
        out_specs=pl.BlockSpec((_BLOCK_ROWS, _N_CLASSES), lambda i: (i, 0)),
        out_shape=jax.ShapeDtypeStruct((_N_ROWS, _N_CLASSES), jnp.float32),
    )(input)


# speedup vs baseline: 1.1792x; 1.1792x over previous
import jax
import jax.numpy as jnp
from jax.experimental import pallas as pl

_N_ROWS = 16384
_N_CLASSES = 1000
_EPS = 0.001
_ON = 1.0 / _N_CLASSES + _EPS
_OFF = 1.0 / _N_CLASSES - _EPS / (_N_CLASSES - 1)

_BLOCK_ROWS = 1024


def _body(x_ref, o_ref):
    x = x_ref[...]
    rowmax = jnp.max(x, axis=1, keepdims=True)
    cols = jax.lax.broadcasted_iota(jnp.int32, x.shape, 1)
    big = jnp.int32(_N_CLASSES)
    amax = jnp.min(jnp.where(x == rowmax, cols, big), axis=1, keepdims=True)
    o_ref[...] = jnp.where(cols == amax, jnp.float32(_ON), jnp.float32(_OFF))


def kernel(input):
    grid = _N_ROWS // _BLOCK_ROWS
    return pl.pallas_call(
        _body,
        grid=(grid,),
        in_specs=[pl.BlockSpec((_BLOCK_ROWS, _N_CLASSES), lambda i: (i, 0))],
        out_specs=pl.BlockSpec((_BLOCK_ROWS, _N_CLASSES), lambda i: (i, 0)),
        out_shape=jax.ShapeDtypeStruct((_N_ROWS, _N_CLASSES), jnp.float32),
    )(input)


# speedup vs baseline: 1.1825x; 1.0028x over previous
import jax
import jax.numpy as jnp
from jax.experimental import pallas as pl

_N_ROWS = 16384
_N_CLASSES = 1000
_EPS = 0.001
_ON = 1.0 / _N_CLASSES + _EPS
_OFF = 1.0 / _N_CLASSES - _EPS / (_N_CLASSES - 1)

_BLOCK_ROWS = 2048


def _body(x_ref, o_ref):
    x = x_ref[...]
    rowmax = jnp.max(x, axis=1, keepdims=True)
    cols = jax.lax.broadcasted_iota(jnp.int32, x.shape, 1)
    big = jnp.int32(_N_CLASSES)
    amax = jnp.min(jnp.where(x == rowmax, cols, big), axis=1, keepdims=True)
    o_ref[...] = jnp.where(cols == amax, jnp.float32(_ON), jnp.float32(_OFF))


def kernel(input):
    grid = _N_ROWS // _BLOCK_ROWS
    return pl.pallas_call(
        _body,
        grid=(grid,),
        in_specs=[pl.BlockSpec((_BLOCK_ROWS, _N_CLASSES), lambda i: (i, 0))],
        out_specs=pl.BlockSpec((_BLOCK_ROWS, _N_CLASSES), lambda i: (i, 0)),
        out_shape=jax.ShapeDtypeStruct((_N_ROWS, _N_CLASSES), jnp.float32),
    )(input)
